# baseline (speedup 1.0000x reference)
import functools

import jax
import jax.numpy as jnp
from jax import lax
from jax.experimental import pallas as pl
from jax.experimental.pallas import tpu as pltpu
from jax.experimental.pallas import tpu_sc as plsc

_LANES = 16
_NCHUNK = 2


def _sc_lookup(table_pad, z, charge):
    n = z.shape[0]
    info = plsc.get_sparse_core_info()
    nw = info.num_cores * info.num_subcores
    per_w = n // nw
    chunk = per_w // _NCHUNK
    tpad = table_pad.shape[0]
    mesh = plsc.VectorSubcoreMesh(core_axis_name="c", subcore_axis_name="s")

    @functools.partial(
        pl.kernel,
        mesh=mesh,
        out_type=jax.ShapeDtypeStruct((n,), jnp.float32),
        compiler_params=pltpu.CompilerParams(needs_layout_passes=False),
        scratch_types=[
            pltpu.VMEM((tpad,), jnp.float32),
            pltpu.VMEM((per_w,), jnp.int32),
            pltpu.VMEM((per_w,), jnp.int32),
            pltpu.VMEM((per_w,), jnp.float32),
            pltpu.SemaphoreType.DMA,
        ]
        + [pltpu.SemaphoreType.DMA] * _NCHUNK,
    )
    def k(table_hbm, z_hbm, q_hbm, out_hbm, t_v, z_v, q_v, o_v, sem_o, *sems):
        wid = lax.axis_index("s") * info.num_cores + lax.axis_index("c")
        base = wid * per_w

        cps = []
        for g in range(_NCHUNK):
            lo = g * chunk
            cps.append(
                (
                    pltpu.async_copy(
                        z_hbm.at[pl.ds(base + lo, chunk)],
                        z_v.at[pl.ds(lo, chunk)],
                        sems[g],
                    ),
                    pltpu.async_copy(
                        q_hbm.at[pl.ds(base + lo, chunk)],
                        q_v.at[pl.ds(lo, chunk)],
                        sems[g],
                    ),
                )
            )
        pltpu.sync_copy(table_hbm, t_v)

        cp_o = []
        for g in range(_NCHUNK):
            lo = g * chunk
            cps[g][0].wait()
            cps[g][1].wait()

            @plsc.parallel_loop(lo, lo + chunk, _LANES, unroll=16)
            def body(off):
                z16 = z_v[pl.ds(off, _LANES)]
                q16 = q_v[pl.ds(off, _LANES)]
                idx = z16 * 3 + q16
                o_v[pl.ds(off, _LANES)] = plsc.load_gather(t_v, [idx])

            cp_o.append(
                pltpu.async_copy(
                    o_v.at[pl.ds(lo, chunk)],
                    out_hbm.at[pl.ds(base + lo, chunk)],
                    sem_o,
                )
            )
        for cp in cp_o:
            cp.wait()

    return k(table_pad, z, charge)


def kernel(z, charge, energy_table):
    table_pad = jnp.zeros((128,), jnp.float32).at[:108].set(
        energy_table.reshape(-1)
    )
    return _sc_lookup(table_pad, z, charge)


# speedup vs baseline: 1.0058x; 1.0058x over previous
import functools

import jax
import jax.numpy as jnp
from jax import lax
from jax.experimental import pallas as pl
from jax.experimental.pallas import tpu as pltpu
from jax.experimental.pallas import tpu_sc as plsc

_LANES = 16
_NCHUNK = 2


def _sc_lookup(table_flat, z, charge):
    n = z.shape[0]
    info = plsc.get_sparse_core_info()
    nw = info.num_cores * info.num_subcores
    per_w = n // nw
    chunk = per_w // _NCHUNK
    tflat = table_flat.shape[0]
    mesh = plsc.VectorSubcoreMesh(core_axis_name="c", subcore_axis_name="s")

    @functools.partial(
        pl.kernel,
        mesh=mesh,
        out_type=jax.ShapeDtypeStruct((n,), jnp.float32),
        compiler_params=pltpu.CompilerParams(needs_layout_passes=False),
        scratch_types=[
            pltpu.VMEM((tflat,), jnp.float32),
            pltpu.VMEM((per_w,), jnp.int32),
            pltpu.VMEM((per_w,), jnp.int32),
            pltpu.VMEM((per_w,), jnp.float32),
            pltpu.SemaphoreType.DMA,
        ]
        + [pltpu.SemaphoreType.DMA] * _NCHUNK,
    )
    def k(table_hbm, z_hbm, q_hbm, out_hbm, t_v, z_v, q_v, o_v, sem_o, *sems):
        wid = lax.axis_index("s") * info.num_cores + lax.axis_index("c")
        base = wid * per_w

        cps = []
        for g in range(_NCHUNK):
            lo = g * chunk
            cps.append(
                (
                    pltpu.async_copy(
                        z_hbm.at[pl.ds(base + lo, chunk)],
                        z_v.at[pl.ds(lo, chunk)],
                        sems[g],
                    ),
                    pltpu.async_copy(
                        q_hbm.at[pl.ds(base + lo, chunk)],
                        q_v.at[pl.ds(lo, chunk)],
                        sems[g],
                    ),
                )
            )
        pltpu.sync_copy(table_hbm, t_v)

        cp_o = []
        for g in range(_NCHUNK):
            lo = g * chunk
            cps[g][0].wait()
            cps[g][1].wait()

            @plsc.parallel_loop(lo, lo + chunk, _LANES, unroll=16)
            def body(off):
                z16 = z_v[pl.ds(off, _LANES)]
                q16 = q_v[pl.ds(off, _LANES)]
                idx = z16 * 3 + q16
                o_v[pl.ds(off, _LANES)] = plsc.load_gather(t_v, [idx])

            cp_o.append(
                pltpu.async_copy(
                    o_v.at[pl.ds(lo, chunk)],
                    out_hbm.at[pl.ds(base + lo, chunk)],
                    sem_o,
                )
            )
        for cp in cp_o:
            cp.wait()

    return k(table_flat, z, charge)


def kernel(z, charge, energy_table):
    return _sc_lookup(energy_table.reshape(-1), z, charge)
